# Initial kernel scaffold; baseline (speedup 1.0000x reference)
import jax
import jax.numpy as jnp
from jax.experimental import pallas as pl


def kernel(x, index):
    raise NotImplementedError("write your pallas kernel here")


# speedup vs baseline: 1.6861x; 1.6861x over previous
import functools

import jax
import jax.numpy as jnp
from jax import lax
from jax.experimental import pallas as pl
from jax.experimental.pallas import tpu as pltpu
from jax.experimental.pallas import tpu_sc as plsc

B, S, D = 4, 8192, 1024
DT = 8
SCHUNK = 1024
NDJ = D // DT
NTASK = B * NDJ
NWORKERS = 32
TPW = NTASK // NWORKERS


def _body(x_hbm, idx_hbm, out_hbm, xbuf, ibuf, obuf):
    nc = 2
    wid = lax.axis_index("s") * nc + lax.axis_index("c")

    lane = jnp.arange(16, dtype=jnp.int32)
    col = lane & 7
    half = lane >> 3

    @pl.loop(0, TPW)
    def _task(t):
        g = wid * TPW + t
        b = g // NDJ
        d0 = (g % NDJ) * DT
        pltpu.sync_copy(x_hbm.at[b, :, pl.ds(d0, DT)], xbuf)

        @pl.loop(0, S // SCHUNK)
        def _chunk(c):
            s0 = c * SCHUNK
            pltpu.sync_copy(idx_hbm.at[b, pl.ds(s0, SCHUNK), pl.ds(d0, DT)], ibuf)

            @plsc.parallel_loop(0, SCHUNK // 2, unroll=4)
            def _rows(i):
                rows = half + 2 * i
                idxv = plsc.load_gather(ibuf, [rows, col])
                xv = plsc.load_gather(xbuf, [idxv, col])
                plsc.store_scatter(obuf, [rows, col], xv)

            pltpu.sync_copy(obuf, out_hbm.at[b, pl.ds(s0, SCHUNK), pl.ds(d0, DT)])


@jax.jit
def _index_pool(x, index):
    mesh = plsc.VectorSubcoreMesh(core_axis_name="c", subcore_axis_name="s")
    return pl.kernel(
        _body,
        out_type=jax.ShapeDtypeStruct((B, S, D), jnp.float32),
        mesh=mesh,
        compiler_params=pltpu.CompilerParams(
            use_tc_tiling_on_sc=False, needs_layout_passes=False
        ),
        scratch_types=[
            pltpu.VMEM((S, DT), jnp.float32),
            pltpu.VMEM((SCHUNK, DT), jnp.int32),
            pltpu.VMEM((SCHUNK, DT), jnp.float32),
        ],
    )(x, index)


def kernel(x, index):
    if index.dtype != jnp.int32:
        index = index.astype(jnp.int32)
    return _index_pool(x, index)


# speedup vs baseline: 3.9228x; 2.3265x over previous
import functools

import jax
import jax.numpy as jnp
from jax import lax
from jax.experimental import pallas as pl
from jax.experimental.pallas import tpu as pltpu
from jax.experimental.pallas import tpu_sc as plsc

B, S, D = 4, 8192, 1024
DT = 8
SCHUNK = 1024
NCHUNK = S // SCHUNK
NDJ = D // DT
NTASK = B * NDJ
NWORKERS = 32
TPW = NTASK // NWORKERS


def _body(x_hbm, idx_hbm, out_hbm, xbuf, ib0, ib1, ob0, ob1,
          sem_x, si0, si1, so0, so1):
    nc = 2
    wid = lax.axis_index("s") * nc + lax.axis_index("c")

    lane = jnp.arange(16, dtype=jnp.int32)
    col = lane & 7
    half = lane >> 3

    ibufs = (ib0, ib1)
    obufs = (ob0, ob1)
    isems = (si0, si1)
    osems = (so0, so1)

    @pl.loop(0, TPW)
    def _task(t):
        g = wid * TPW + t
        b = g // NDJ
        d0 = (g % NDJ) * DT

        def idx_slice(c):
            return idx_hbm.at[b, pl.ds(c * SCHUNK, SCHUNK), pl.ds(d0, DT)]

        def out_slice(c):
            return out_hbm.at[b, pl.ds(c * SCHUNK, SCHUNK), pl.ds(d0, DT)]

        cp_x = pltpu.async_copy(x_hbm.at[b, :, pl.ds(d0, DT)], xbuf, sem_x)
        pltpu.async_copy(idx_slice(0), ibufs[0], isems[0])
        cp_x.wait()

        for c in range(NCHUNK):
            p = c % 2
            ib, ob = ibufs[p], obufs[p]
            pltpu.make_async_copy(idx_slice(c), ib, isems[p]).wait()
            if c + 1 < NCHUNK:
                pltpu.async_copy(idx_slice(c + 1), ibufs[(c + 1) % 2],
                                 isems[(c + 1) % 2])
            if c >= 2:
                pltpu.make_async_copy(ob, out_slice(c - 2), osems[p]).wait()

            @plsc.parallel_loop(0, SCHUNK // 2, unroll=8)
            def _rows(i):
                rows = half + 2 * i
                idxv = plsc.load_gather(ib, [rows, col])
                xv = plsc.load_gather(xbuf, [idxv, col])
                plsc.store_scatter(ob, [rows, col], xv)

            pltpu.async_copy(ob, out_slice(c), osems[p])

        pltpu.make_async_copy(obufs[0], out_slice(NCHUNK - 2), osems[0]).wait()
        pltpu.make_async_copy(obufs[1], out_slice(NCHUNK - 1), osems[1]).wait()


@jax.jit
def _index_pool(x, index):
    mesh = plsc.VectorSubcoreMesh(core_axis_name="c", subcore_axis_name="s")
    return pl.kernel(
        _body,
        out_type=jax.ShapeDtypeStruct((B, S, D), jnp.float32),
        mesh=mesh,
        compiler_params=pltpu.CompilerParams(
            use_tc_tiling_on_sc=False, needs_layout_passes=False
        ),
        scratch_types=[
            pltpu.VMEM((S, DT), jnp.float32),
            pltpu.VMEM((SCHUNK, DT), jnp.int32),
            pltpu.VMEM((SCHUNK, DT), jnp.int32),
            pltpu.VMEM((SCHUNK, DT), jnp.float32),
            pltpu.VMEM((SCHUNK, DT), jnp.float32),
            pltpu.SemaphoreType.DMA,
            pltpu.SemaphoreType.DMA,
            pltpu.SemaphoreType.DMA,
            pltpu.SemaphoreType.DMA,
            pltpu.SemaphoreType.DMA,
        ],
    )(x, index)


def kernel(x, index):
    if index.dtype != jnp.int32:
        index = index.astype(jnp.int32)
    return _index_pool(x, index)
